# baseline (speedup 1.0000x reference)
import functools

import jax
import jax.numpy as jnp
from jax import lax
from jax.experimental import pallas as pl
from jax.experimental.pallas import tpu as pltpu
from jax.experimental.pallas import tpu_sc as plsc

GRID = 16
V = GRID ** 3
NF = 10
EPS = 1e-6

NC = 2
NS = 16
L = 16
NW = NC * NS

DB_IN = 8000
DB_OUT = 8192
PB = 1024


def _depad_body(nblk, x_ref, xs_ref, ys_ref, zs_ref, par_ref, mn_ref, mx_ref):
    i = pl.program_id(0)
    blk = x_ref[...]
    bmin = jnp.min(blk, axis=0, keepdims=True)
    bmax = jnp.max(blk, axis=0, keepdims=True)
    pad = jnp.zeros((DB_OUT - DB_IN, 3), jnp.float32)
    t = jnp.concatenate([blk, pad], axis=0).T
    xs_ref[...] = t[0]
    ys_ref[...] = t[1]
    zs_ref[...] = t[2]

    @pl.when(i == 0)
    def _():
        mn_ref[...] = bmin
        mx_ref[...] = bmax

    @pl.when(i != 0)
    def _():
        mn_ref[...] = jnp.minimum(mn_ref[...], bmin)
        mx_ref[...] = jnp.maximum(mx_ref[...], bmax)

    @pl.when(i == nblk - 1)
    def _():
        mins = mn_ref[...]
        scale = GRID / (mx_ref[...] - mins + EPS)
        par = jnp.concatenate(
            [jnp.zeros((1, 1), jnp.float32), mins, scale,
             jnp.zeros((1, PB - 7), jnp.float32)], axis=1)
        par_ref[...] = par.reshape(PB)


def _depad(x):
    n = x.shape[0]
    assert n % DB_IN == 0
    nblk = n // DB_IN
    np_out = nblk * DB_OUT
    return pl.pallas_call(
        functools.partial(_depad_body, nblk),
        grid=(nblk,),
        in_specs=[pl.BlockSpec((DB_IN, 3), lambda i: (i, 0))],
        out_specs=[pl.BlockSpec((DB_OUT,), lambda i: (i,)),
                   pl.BlockSpec((DB_OUT,), lambda i: (i,)),
                   pl.BlockSpec((DB_OUT,), lambda i: (i,)),
                   pl.BlockSpec((PB,), lambda i: (0,))],
        out_shape=[jax.ShapeDtypeStruct((np_out,), jnp.float32),
                   jax.ShapeDtypeStruct((np_out,), jnp.float32),
                   jax.ShapeDtypeStruct((np_out,), jnp.float32),
                   jax.ShapeDtypeStruct((PB,), jnp.float32)],
        scratch_shapes=[pltpu.VMEM((1, 3), jnp.float32),
                        pltpu.VMEM((1, 3), jnp.float32)],
    )(x)


@functools.lru_cache(maxsize=None)
def _make_scatter(n_slots):
    assert n_slots % DB_OUT == 0
    n_chunks = n_slots // DB_OUT
    max_ch_per_w = (n_chunks + NW - 1) // NW
    groups = DB_IN // (2 * L)

    mesh = plsc.VectorSubcoreMesh(
        core_axis_name="c", subcore_axis_name="s",
        num_cores=NC, num_subcores=NS)

    @functools.partial(
        pl.kernel,
        mesh=mesh,
        compiler_params=pltpu.CompilerParams(needs_layout_passes=False),
        out_type=jax.ShapeDtypeStruct((NW, NF * V), jnp.float32),
        scratch_types=[
            pltpu.VMEM((DB_OUT,), jnp.float32),
            pltpu.VMEM((DB_OUT,), jnp.float32),
            pltpu.VMEM((DB_OUT,), jnp.float32),
            pltpu.VMEM((NF * V,), jnp.float32),
            pltpu.VMEM((L,), jnp.float32),
        ],
    )
    def scatter(xs, ys, zs, params, parts, bx, by, bz, acc, pv):
        wid = lax.axis_index("s") * NC + lax.axis_index("c")
        pltpu.sync_copy(params.at[pl.ds(0, L)], pv)

        zero16 = jnp.zeros((L,), jnp.float32)

        def zbody(i, c):
            acc[pl.ds(i * L, L)] = zero16
            return c
        lax.fori_loop(0, NF * V // L, zbody, 0)

        idx0 = jnp.zeros((L,), jnp.int32)
        m0 = plsc.load_gather(pv, [idx0 + 1])
        m1 = plsc.load_gather(pv, [idx0 + 2])
        m2 = plsc.load_gather(pv, [idx0 + 3])
        s0 = plsc.load_gather(pv, [idx0 + 4])
        s1 = plsc.load_gather(pv, [idx0 + 5])
        s2 = plsc.load_gather(pv, [idx0 + 6])

        ones = jnp.ones((L,), jnp.float32)

        def do_group(o):
            xv = bx[pl.ds(o, L)]
            yv = by[pl.ds(o, L)]
            zv = bz[pl.ds(o, L)]
            fx = jnp.clip(((xv - m0) * s0).astype(jnp.int32), 0, GRID - 1)
            fy = jnp.clip(((yv - m1) * s1).astype(jnp.int32), 0, GRID - 1)
            fz = jnp.clip(((zv - m2) * s2).astype(jnp.int32), 0, GRID - 1)
            vid = (fx * GRID + fy) * GRID + fz
            plsc.addupdate_scatter(acc, [vid], ones)
            plsc.addupdate_scatter(acc, [vid + V], xv)
            plsc.addupdate_scatter(acc, [vid + 2 * V], yv)
            plsc.addupdate_scatter(acc, [vid + 3 * V], zv)
            plsc.addupdate_scatter(acc, [vid + 4 * V], xv * xv)
            plsc.addupdate_scatter(acc, [vid + 5 * V], xv * yv)
            plsc.addupdate_scatter(acc, [vid + 6 * V], xv * zv)
            plsc.addupdate_scatter(acc, [vid + 7 * V], yv * yv)
            plsc.addupdate_scatter(acc, [vid + 8 * V], yv * zv)
            plsc.addupdate_scatter(acc, [vid + 9 * V], zv * zv)

        def chunk_body(k, c):
            ci = wid + k * NW

            @pl.when(ci < n_chunks)
            def _():
                base = ci * DB_OUT
                pltpu.sync_copy(xs.at[pl.ds(base, DB_OUT)], bx)
                pltpu.sync_copy(ys.at[pl.ds(base, DB_OUT)], by)
                pltpu.sync_copy(zs.at[pl.ds(base, DB_OUT)], bz)

                def gbody(g, cc):
                    o = g * (2 * L)
                    do_group(o)
                    do_group(o + L)
                    return cc
                lax.fori_loop(0, groups, gbody, 0)
            return c
        lax.fori_loop(0, max_ch_per_w, chunk_body, 0)

        pltpu.sync_copy(acc, parts.at[wid])

    return scatter


def _fin_body(p_ref, mean_ref, cov_ref):
    t = jnp.sum(p_ref[...], axis=0)
    cnt = t[0:1]
    denom = jnp.maximum(cnt, 1.0)
    mu = t[1:4] / denom
    sec = t[4:10] / denom
    mean_ref[...] = mu
    mx, my, mz = mu[0:1], mu[1:2], mu[2:3]
    c00 = sec[0:1] - mx * mx
    c01 = sec[1:2] - mx * my
    c02 = sec[2:3] - mx * mz
    c11 = sec[3:4] - my * my
    c12 = sec[4:5] - my * mz
    c22 = sec[5:6] - mz * mz
    cov_ref[...] = jnp.concatenate(
        [c00, c01, c02, c01, c11, c12, c02, c12, c22], axis=0)


def _finalize(parts3):
    return pl.pallas_call(
        _fin_body,
        out_shape=[jax.ShapeDtypeStruct((3, V), jnp.float32),
                   jax.ShapeDtypeStruct((9, V), jnp.float32)],
    )(parts3)


def kernel(x):
    xs, ys, zs, params = _depad(x)

    parts = _make_scatter(xs.shape[0])(xs, ys, zs, params)

    mean_t, cov_t = _finalize(parts.reshape(NW, NF, V))
    means = mean_t.T
    covs = cov_t.T.reshape(V, 3, 3)
    return means, covs


# speedup vs baseline: 1.0707x; 1.0707x over previous
import functools

import jax
import jax.numpy as jnp
from jax import lax
from jax.experimental import pallas as pl
from jax.experimental.pallas import tpu as pltpu
from jax.experimental.pallas import tpu_sc as plsc

GRID = 16
V = GRID ** 3
NF = 10
EPS = 1e-6

NC = 2
NS = 16
L = 16
NW = NC * NS

DB_IN = 8000
DB_OUT = 8192
PB = 1024


def _plane_split(blk):
    pad = jnp.zeros((DB_OUT - DB_IN, 3), jnp.float32)
    t = jnp.concatenate([blk, pad], axis=0).T
    return t


def _depad_body(nhalf, xa_ref, xb_ref, xsa_ref, ysa_ref, zsa_ref,
                xsb_ref, ysb_ref, zsb_ref, par_ref, mn_ref, mx_ref):
    i = pl.program_id(0)
    blka = xa_ref[...]
    blkb = xb_ref[...]
    bmin = jnp.minimum(jnp.min(blka, axis=0, keepdims=True),
                       jnp.min(blkb, axis=0, keepdims=True))
    bmax = jnp.maximum(jnp.max(blka, axis=0, keepdims=True),
                       jnp.max(blkb, axis=0, keepdims=True))
    ta = _plane_split(blka)
    tb = _plane_split(blkb)
    xsa_ref[...] = ta[0]
    ysa_ref[...] = ta[1]
    zsa_ref[...] = ta[2]
    xsb_ref[...] = tb[0]
    ysb_ref[...] = tb[1]
    zsb_ref[...] = tb[2]

    @pl.when(i == 0)
    def _():
        mn_ref[...] = bmin
        mx_ref[...] = bmax

    @pl.when(i != 0)
    def _():
        mn_ref[...] = jnp.minimum(mn_ref[...], bmin)
        mx_ref[...] = jnp.maximum(mx_ref[...], bmax)

    @pl.when(i == nhalf - 1)
    def _():
        mins = mn_ref[...]
        scale = GRID / (mx_ref[...] - mins + EPS)
        par = jnp.concatenate(
            [jnp.zeros((1, 1), jnp.float32), mins, scale,
             jnp.zeros((1, PB - 7), jnp.float32)], axis=1)
        par_ref[...] = par.reshape(PB)


def _depad(x):
    n = x.shape[0]
    assert n % (2 * DB_IN) == 0
    nhalf = n // (2 * DB_IN)
    np_half = nhalf * DB_OUT
    pp = jax.ShapeDtypeStruct((np_half,), jnp.float32)
    return pl.pallas_call(
        functools.partial(_depad_body, nhalf),
        grid=(nhalf,),
        in_specs=[pl.BlockSpec((DB_IN, 3), lambda i: (i, 0)),
                  pl.BlockSpec((DB_IN, 3), lambda i, nh=nhalf: (i + nh, 0))],
        out_specs=[pl.BlockSpec((DB_OUT,), lambda i: (i,)),
                   pl.BlockSpec((DB_OUT,), lambda i: (i,)),
                   pl.BlockSpec((DB_OUT,), lambda i: (i,)),
                   pl.BlockSpec((DB_OUT,), lambda i: (i,)),
                   pl.BlockSpec((DB_OUT,), lambda i: (i,)),
                   pl.BlockSpec((DB_OUT,), lambda i: (i,)),
                   pl.BlockSpec((PB,), lambda i: (0,))],
        out_shape=[pp, pp, pp, pp, pp, pp,
                   jax.ShapeDtypeStruct((PB,), jnp.float32)],
        scratch_shapes=[pltpu.VMEM((1, 3), jnp.float32),
                        pltpu.VMEM((1, 3), jnp.float32)],
    )(x, x)


@functools.lru_cache(maxsize=None)
def _make_scatter(n_slots):
    assert n_slots % DB_OUT == 0
    n_chunks = n_slots // DB_OUT
    max_ch_per_w = (n_chunks + NW - 1) // NW
    groups = DB_IN // (2 * L)

    mesh = plsc.VectorSubcoreMesh(
        core_axis_name="c", subcore_axis_name="s",
        num_cores=NC, num_subcores=NS)

    @functools.partial(
        pl.kernel,
        mesh=mesh,
        compiler_params=pltpu.CompilerParams(needs_layout_passes=False),
        out_type=jax.ShapeDtypeStruct((NW, NF * V), jnp.float32),
        scratch_types=[
            pltpu.VMEM((DB_OUT,), jnp.float32),
            pltpu.VMEM((DB_OUT,), jnp.float32),
            pltpu.VMEM((DB_OUT,), jnp.float32),
            pltpu.VMEM((NF * V,), jnp.float32),
            pltpu.VMEM((L,), jnp.float32),
        ],
    )
    def scatter(xsa, ysa, zsa, xsb, ysb, zsb, params, parts, bx, by, bz, acc, pv):
        wid = lax.axis_index("s") * NC + lax.axis_index("c")
        pltpu.sync_copy(params.at[pl.ds(0, L)], pv)

        zero16 = jnp.zeros((L,), jnp.float32)

        def zbody(i, c):
            acc[pl.ds(i * L, L)] = zero16
            return c
        lax.fori_loop(0, NF * V // L, zbody, 0)

        idx0 = jnp.zeros((L,), jnp.int32)
        m0 = plsc.load_gather(pv, [idx0 + 1])
        m1 = plsc.load_gather(pv, [idx0 + 2])
        m2 = plsc.load_gather(pv, [idx0 + 3])
        s0 = plsc.load_gather(pv, [idx0 + 4])
        s1 = plsc.load_gather(pv, [idx0 + 5])
        s2 = plsc.load_gather(pv, [idx0 + 6])

        ones = jnp.ones((L,), jnp.float32)

        def do_group(o):
            xv = bx[pl.ds(o, L)]
            yv = by[pl.ds(o, L)]
            zv = bz[pl.ds(o, L)]
            fx = jnp.clip(((xv - m0) * s0).astype(jnp.int32), 0, GRID - 1)
            fy = jnp.clip(((yv - m1) * s1).astype(jnp.int32), 0, GRID - 1)
            fz = jnp.clip(((zv - m2) * s2).astype(jnp.int32), 0, GRID - 1)
            vid = (fx * GRID + fy) * GRID + fz
            plsc.addupdate_scatter(acc, [vid], ones)
            plsc.addupdate_scatter(acc, [vid + V], xv)
            plsc.addupdate_scatter(acc, [vid + 2 * V], yv)
            plsc.addupdate_scatter(acc, [vid + 3 * V], zv)
            plsc.addupdate_scatter(acc, [vid + 4 * V], xv * xv)
            plsc.addupdate_scatter(acc, [vid + 5 * V], xv * yv)
            plsc.addupdate_scatter(acc, [vid + 6 * V], xv * zv)
            plsc.addupdate_scatter(acc, [vid + 7 * V], yv * yv)
            plsc.addupdate_scatter(acc, [vid + 8 * V], yv * zv)
            plsc.addupdate_scatter(acc, [vid + 9 * V], zv * zv)

        def chunk_body(k, c):
            ci = wid + k * NW

            @pl.when(ci < n_chunks)
            def _():
                nh = n_chunks // 2

                @pl.when(ci < nh)
                def _():
                    base = ci * DB_OUT
                    pltpu.sync_copy(xsa.at[pl.ds(base, DB_OUT)], bx)
                    pltpu.sync_copy(ysa.at[pl.ds(base, DB_OUT)], by)
                    pltpu.sync_copy(zsa.at[pl.ds(base, DB_OUT)], bz)

                @pl.when(ci >= nh)
                def _():
                    base = (ci - nh) * DB_OUT
                    pltpu.sync_copy(xsb.at[pl.ds(base, DB_OUT)], bx)
                    pltpu.sync_copy(ysb.at[pl.ds(base, DB_OUT)], by)
                    pltpu.sync_copy(zsb.at[pl.ds(base, DB_OUT)], bz)

                def gbody(g, cc):
                    o = g * (2 * L)
                    do_group(o)
                    do_group(o + L)
                    return cc
                lax.fori_loop(0, groups, gbody, 0)
            return c
        lax.fori_loop(0, max_ch_per_w, chunk_body, 0)

        pltpu.sync_copy(acc, parts.at[wid])

    return scatter


def _fin_body(p_ref, mean_ref, cov_ref):
    t = jnp.sum(p_ref[...], axis=0)
    cnt = t[0:1]
    denom = jnp.maximum(cnt, 1.0)
    mu = t[1:4] / denom
    sec = t[4:10] / denom
    mean_ref[...] = mu
    mx, my, mz = mu[0:1], mu[1:2], mu[2:3]
    c00 = sec[0:1] - mx * mx
    c01 = sec[1:2] - mx * my
    c02 = sec[2:3] - mx * mz
    c11 = sec[3:4] - my * my
    c12 = sec[4:5] - my * mz
    c22 = sec[5:6] - mz * mz
    cov_ref[...] = jnp.concatenate(
        [c00, c01, c02, c01, c11, c12, c02, c12, c22], axis=0)


def _finalize(parts3):
    return pl.pallas_call(
        _fin_body,
        out_shape=[jax.ShapeDtypeStruct((3, V), jnp.float32),
                   jax.ShapeDtypeStruct((9, V), jnp.float32)],
    )(parts3)


def kernel(x):
    xsa, ysa, zsa, xsb, ysb, zsb, params = _depad(x)

    parts = _make_scatter(2 * xsa.shape[0])(
        xsa, ysa, zsa, xsb, ysb, zsb, params)

    mean_t, cov_t = _finalize(parts.reshape(NW, NF, V))
    means = mean_t.T
    covs = cov_t.T.reshape(V, 3, 3)
    return means, covs


# speedup vs baseline: 1.1103x; 1.0370x over previous
import functools

import jax
import jax.numpy as jnp
from jax import lax
from jax.experimental import pallas as pl
from jax.experimental.pallas import tpu as pltpu
from jax.experimental.pallas import tpu_sc as plsc

GRID = 16
V = GRID ** 3
NF = 10
EPS = 1e-6

NC = 2
NS = 16
L = 16
NW = NC * NS

DB_IN = 8000
DB_OUT = 8192
PB = 1024
NSTR = 5


def _plane_split(blk):
    pad = jnp.zeros((DB_OUT - DB_IN, 3), jnp.float32)
    return jnp.concatenate([blk, pad], axis=0).T


def _depad_body(nblk, *refs):
    xrefs = refs[:NSTR]
    orefs = refs[NSTR:NSTR + 3 * NSTR]
    par_ref = refs[NSTR + 3 * NSTR]
    mn_ref, mx_ref = refs[-2:]
    i = pl.program_id(0)

    blks = [r[...] for r in xrefs]
    bmin = blks[0].min(axis=0, keepdims=True)
    bmax = blks[0].max(axis=0, keepdims=True)
    for b in blks[1:]:
        bmin = jnp.minimum(bmin, b.min(axis=0, keepdims=True))
        bmax = jnp.maximum(bmax, b.max(axis=0, keepdims=True))
    for s, b in enumerate(blks):
        t = _plane_split(b)
        orefs[3 * s][...] = t[0]
        orefs[3 * s + 1][...] = t[1]
        orefs[3 * s + 2][...] = t[2]

    @pl.when(i == 0)
    def _():
        mn_ref[...] = bmin
        mx_ref[...] = bmax

    @pl.when(i != 0)
    def _():
        mn_ref[...] = jnp.minimum(mn_ref[...], bmin)
        mx_ref[...] = jnp.maximum(mx_ref[...], bmax)

    @pl.when(i == nblk - 1)
    def _():
        mins = mn_ref[...]
        scale = GRID / (mx_ref[...] - mins + EPS)
        par = jnp.concatenate(
            [jnp.zeros((1, 1), jnp.float32), mins, scale,
             jnp.zeros((1, PB - 7), jnp.float32)], axis=1)
        par_ref[...] = par.reshape(PB)


def _depad(x):
    n = x.shape[0]
    assert n % (NSTR * DB_IN) == 0
    nblk = n // (NSTR * DB_IN)
    np_s = nblk * DB_OUT
    pp = jax.ShapeDtypeStruct((np_s,), jnp.float32)
    in_specs = [pl.BlockSpec((DB_IN, 3), lambda i, s=s: (i + s * nblk, 0))
                for s in range(NSTR)]
    out_specs = ([pl.BlockSpec((DB_OUT,), lambda i: (i,))] * (3 * NSTR)
                 + [pl.BlockSpec((PB,), lambda i: (0,))])
    return pl.pallas_call(
        functools.partial(_depad_body, nblk),
        grid=(nblk,),
        in_specs=in_specs,
        out_specs=out_specs,
        out_shape=[pp] * (3 * NSTR) + [jax.ShapeDtypeStruct((PB,), jnp.float32)],
        scratch_shapes=[pltpu.VMEM((1, 3), jnp.float32),
                        pltpu.VMEM((1, 3), jnp.float32)],
    )(*([x] * NSTR))


@functools.lru_cache(maxsize=None)
def _make_scatter(n_slots):
    assert n_slots % (NSTR * DB_OUT) == 0
    n_chunks = n_slots // DB_OUT
    nper = n_chunks // NSTR
    max_ch_per_w = (n_chunks + NW - 1) // NW
    groups = DB_IN // (2 * L)

    mesh = plsc.VectorSubcoreMesh(
        core_axis_name="c", subcore_axis_name="s",
        num_cores=NC, num_subcores=NS)

    @functools.partial(
        pl.kernel,
        mesh=mesh,
        compiler_params=pltpu.CompilerParams(needs_layout_passes=False),
        out_type=jax.ShapeDtypeStruct((NW, NF * V), jnp.float32),
        scratch_types=[
            pltpu.VMEM((DB_OUT,), jnp.float32),
            pltpu.VMEM((DB_OUT,), jnp.float32),
            pltpu.VMEM((DB_OUT,), jnp.float32),
            pltpu.VMEM((NF * V,), jnp.float32),
            pltpu.VMEM((L,), jnp.float32),
        ],
    )
    def scatter(*refs):
        planes = refs[:3 * NSTR]
        params = refs[3 * NSTR]
        parts = refs[3 * NSTR + 1]
        bx, by, bz, acc, pv = refs[3 * NSTR + 2:]

        wid = lax.axis_index("s") * NC + lax.axis_index("c")
        pltpu.sync_copy(params.at[pl.ds(0, L)], pv)

        zero16 = jnp.zeros((L,), jnp.float32)

        def zbody(i, c):
            acc[pl.ds(i * L, L)] = zero16
            return c
        lax.fori_loop(0, NF * V // L, zbody, 0)

        idx0 = jnp.zeros((L,), jnp.int32)
        m0 = plsc.load_gather(pv, [idx0 + 1])
        m1 = plsc.load_gather(pv, [idx0 + 2])
        m2 = plsc.load_gather(pv, [idx0 + 3])
        s0 = plsc.load_gather(pv, [idx0 + 4])
        s1 = plsc.load_gather(pv, [idx0 + 5])
        s2 = plsc.load_gather(pv, [idx0 + 6])

        ones = jnp.ones((L,), jnp.float32)

        def do_group(o):
            xv = bx[pl.ds(o, L)]
            yv = by[pl.ds(o, L)]
            zv = bz[pl.ds(o, L)]
            fx = jnp.clip(((xv - m0) * s0).astype(jnp.int32), 0, GRID - 1)
            fy = jnp.clip(((yv - m1) * s1).astype(jnp.int32), 0, GRID - 1)
            fz = jnp.clip(((zv - m2) * s2).astype(jnp.int32), 0, GRID - 1)
            vid = (fx * GRID + fy) * GRID + fz
            plsc.addupdate_scatter(acc, [vid], ones)
            plsc.addupdate_scatter(acc, [vid + V], xv)
            plsc.addupdate_scatter(acc, [vid + 2 * V], yv)
            plsc.addupdate_scatter(acc, [vid + 3 * V], zv)
            plsc.addupdate_scatter(acc, [vid + 4 * V], xv * xv)
            plsc.addupdate_scatter(acc, [vid + 5 * V], xv * yv)
            plsc.addupdate_scatter(acc, [vid + 6 * V], xv * zv)
            plsc.addupdate_scatter(acc, [vid + 7 * V], yv * yv)
            plsc.addupdate_scatter(acc, [vid + 8 * V], yv * zv)
            plsc.addupdate_scatter(acc, [vid + 9 * V], zv * zv)

        def chunk_body(k, c):
            ci = wid + k * NW

            @pl.when(ci < n_chunks)
            def _():
                for s in range(NSTR):
                    @pl.when((ci >= s * nper) & (ci < (s + 1) * nper))
                    def _(s=s):
                        base = (ci - s * nper) * DB_OUT
                        pltpu.sync_copy(
                            planes[3 * s].at[pl.ds(base, DB_OUT)], bx)
                        pltpu.sync_copy(
                            planes[3 * s + 1].at[pl.ds(base, DB_OUT)], by)
                        pltpu.sync_copy(
                            planes[3 * s + 2].at[pl.ds(base, DB_OUT)], bz)

                def gbody(g, cc):
                    o = g * (2 * L)
                    do_group(o)
                    do_group(o + L)
                    return cc
                lax.fori_loop(0, groups, gbody, 0)
            return c
        lax.fori_loop(0, max_ch_per_w, chunk_body, 0)

        pltpu.sync_copy(acc, parts.at[wid])

    return scatter


def _fin_body(p_ref, mean_ref, cov_ref):
    t = jnp.sum(p_ref[...], axis=0)
    cnt = t[0:1]
    denom = jnp.maximum(cnt, 1.0)
    mu = t[1:4] / denom
    sec = t[4:10] / denom
    mean_ref[...] = mu
    mx, my, mz = mu[0:1], mu[1:2], mu[2:3]
    c00 = sec[0:1] - mx * mx
    c01 = sec[1:2] - mx * my
    c02 = sec[2:3] - mx * mz
    c11 = sec[3:4] - my * my
    c12 = sec[4:5] - my * mz
    c22 = sec[5:6] - mz * mz
    cov_ref[...] = jnp.concatenate(
        [c00, c01, c02, c01, c11, c12, c02, c12, c22], axis=0)


def _finalize(parts3):
    return pl.pallas_call(
        _fin_body,
        out_shape=[jax.ShapeDtypeStruct((3, V), jnp.float32),
                   jax.ShapeDtypeStruct((9, V), jnp.float32)],
    )(parts3)


def kernel(x):
    outs = _depad(x)
    planes, params = outs[:3 * NSTR], outs[3 * NSTR]

    parts = _make_scatter(NSTR * planes[0].shape[0])(*planes, params)

    mean_t, cov_t = _finalize(parts.reshape(NW, NF, V))
    means = mean_t.T
    covs = cov_t.T.reshape(V, 3, 3)
    return means, covs


# speedup vs baseline: 1.1107x; 1.0004x over previous
import functools

import jax
import jax.numpy as jnp
from jax import lax
from jax.experimental import pallas as pl
from jax.experimental.pallas import tpu as pltpu
from jax.experimental.pallas import tpu_sc as plsc

GRID = 16
V = GRID ** 3
NF = 10
EPS = 1e-6

NC = 2
NS = 16
L = 16
NW = NC * NS

DB_IN = 8000
DB_OUT = 8192
PB = 1024
NSTR = 5


def _plane_split(blk):
    pad = jnp.zeros((DB_OUT - DB_IN, 3), jnp.float32)
    return jnp.concatenate([blk, pad], axis=0).T


def _depad_body(nblk, *refs):
    xrefs = refs[:NSTR]
    orefs = refs[NSTR:NSTR + 3 * NSTR]
    par_ref = refs[NSTR + 3 * NSTR]
    mn_ref, mx_ref = refs[-2:]
    i = pl.program_id(0)

    blks = [r[...] for r in xrefs]
    bmin = blks[0].min(axis=0, keepdims=True)
    bmax = blks[0].max(axis=0, keepdims=True)
    for b in blks[1:]:
        bmin = jnp.minimum(bmin, b.min(axis=0, keepdims=True))
        bmax = jnp.maximum(bmax, b.max(axis=0, keepdims=True))
    for s, b in enumerate(blks):
        t = _plane_split(b)
        orefs[3 * s][...] = t[0]
        orefs[3 * s + 1][...] = t[1]
        orefs[3 * s + 2][...] = t[2]

    @pl.when(i == 0)
    def _():
        mn_ref[...] = bmin
        mx_ref[...] = bmax

    @pl.when(i != 0)
    def _():
        mn_ref[...] = jnp.minimum(mn_ref[...], bmin)
        mx_ref[...] = jnp.maximum(mx_ref[...], bmax)

    @pl.when(i == nblk - 1)
    def _():
        mins = mn_ref[...]
        scale = GRID / (mx_ref[...] - mins + EPS)
        par = jnp.concatenate(
            [jnp.zeros((1, 1), jnp.float32), mins, scale,
             jnp.zeros((1, PB - 7), jnp.float32)], axis=1)
        par_ref[...] = par.reshape(PB)


def _depad(x):
    n = x.shape[0]
    assert n % (NSTR * DB_IN) == 0
    nblk = n // (NSTR * DB_IN)
    np_s = nblk * DB_OUT
    pp = jax.ShapeDtypeStruct((np_s,), jnp.float32)
    in_specs = [pl.BlockSpec((DB_IN, 3), lambda i, s=s: (i + s * nblk, 0))
                for s in range(NSTR)]
    out_specs = ([pl.BlockSpec((DB_OUT,), lambda i: (i,))] * (3 * NSTR)
                 + [pl.BlockSpec((PB,), lambda i: (0,))])
    return pl.pallas_call(
        functools.partial(_depad_body, nblk),
        grid=(nblk,),
        in_specs=in_specs,
        out_specs=out_specs,
        out_shape=[pp] * (3 * NSTR) + [jax.ShapeDtypeStruct((PB,), jnp.float32)],
        scratch_shapes=[pltpu.VMEM((1, 3), jnp.float32),
                        pltpu.VMEM((1, 3), jnp.float32)],
    )(*([x] * NSTR))


@functools.lru_cache(maxsize=None)
def _make_scatter(n_slots):
    assert n_slots % (NSTR * DB_OUT) == 0
    n_chunks = n_slots // DB_OUT
    nper = n_chunks // NSTR
    max_ch_per_w = (n_chunks + NW - 1) // NW
    groups = DB_IN // (4 * L)

    mesh = plsc.VectorSubcoreMesh(
        core_axis_name="c", subcore_axis_name="s",
        num_cores=NC, num_subcores=NS)

    @functools.partial(
        pl.kernel,
        mesh=mesh,
        compiler_params=pltpu.CompilerParams(needs_layout_passes=False),
        out_type=jax.ShapeDtypeStruct((NW, NF * V), jnp.float32),
        scratch_types=[
            pltpu.VMEM((DB_OUT,), jnp.float32),
            pltpu.VMEM((DB_OUT,), jnp.float32),
            pltpu.VMEM((DB_OUT,), jnp.float32),
            pltpu.VMEM((NF * V,), jnp.float32),
            pltpu.VMEM((L,), jnp.float32),
        ],
    )
    def scatter(*refs):
        planes = refs[:3 * NSTR]
        params = refs[3 * NSTR]
        parts = refs[3 * NSTR + 1]
        bx, by, bz, acc, pv = refs[3 * NSTR + 2:]

        wid = lax.axis_index("s") * NC + lax.axis_index("c")
        pltpu.sync_copy(params.at[pl.ds(0, L)], pv)

        zero16 = jnp.zeros((L,), jnp.float32)

        def zbody(i, c):
            acc[pl.ds(i * L, L)] = zero16
            return c
        lax.fori_loop(0, NF * V // L, zbody, 0)

        idx0 = jnp.zeros((L,), jnp.int32)
        m0 = plsc.load_gather(pv, [idx0 + 1])
        m1 = plsc.load_gather(pv, [idx0 + 2])
        m2 = plsc.load_gather(pv, [idx0 + 3])
        s0 = plsc.load_gather(pv, [idx0 + 4])
        s1 = plsc.load_gather(pv, [idx0 + 5])
        s2 = plsc.load_gather(pv, [idx0 + 6])

        ones = jnp.ones((L,), jnp.float32)

        def do_group(o):
            xv = bx[pl.ds(o, L)]
            yv = by[pl.ds(o, L)]
            zv = bz[pl.ds(o, L)]
            fx = jnp.clip(((xv - m0) * s0).astype(jnp.int32), 0, GRID - 1)
            fy = jnp.clip(((yv - m1) * s1).astype(jnp.int32), 0, GRID - 1)
            fz = jnp.clip(((zv - m2) * s2).astype(jnp.int32), 0, GRID - 1)
            vid = (fx * GRID + fy) * GRID + fz
            plsc.addupdate_scatter(acc, [vid], ones)
            plsc.addupdate_scatter(acc, [vid + V], xv)
            plsc.addupdate_scatter(acc, [vid + 2 * V], yv)
            plsc.addupdate_scatter(acc, [vid + 3 * V], zv)
            plsc.addupdate_scatter(acc, [vid + 4 * V], xv * xv)
            plsc.addupdate_scatter(acc, [vid + 5 * V], xv * yv)
            plsc.addupdate_scatter(acc, [vid + 6 * V], xv * zv)
            plsc.addupdate_scatter(acc, [vid + 7 * V], yv * yv)
            plsc.addupdate_scatter(acc, [vid + 8 * V], yv * zv)
            plsc.addupdate_scatter(acc, [vid + 9 * V], zv * zv)

        def chunk_body(k, c):
            ci = wid + k * NW

            @pl.when(ci < n_chunks)
            def _():
                for s in range(NSTR):
                    @pl.when((ci >= s * nper) & (ci < (s + 1) * nper))
                    def _(s=s):
                        base = (ci - s * nper) * DB_OUT
                        pltpu.sync_copy(
                            planes[3 * s].at[pl.ds(base, DB_OUT)], bx)
                        pltpu.sync_copy(
                            planes[3 * s + 1].at[pl.ds(base, DB_OUT)], by)
                        pltpu.sync_copy(
                            planes[3 * s + 2].at[pl.ds(base, DB_OUT)], bz)

                def gbody(g, cc):
                    o = g * (4 * L)
                    do_group(o)
                    do_group(o + L)
                    do_group(o + 2 * L)
                    do_group(o + 3 * L)
                    return cc
                lax.fori_loop(0, groups, gbody, 0)
            return c
        lax.fori_loop(0, max_ch_per_w, chunk_body, 0)

        pltpu.sync_copy(acc, parts.at[wid])

    return scatter


def _fin_body(p_ref, mean_ref, cov_ref):
    t = jnp.sum(p_ref[...], axis=0)
    cnt = t[0:1]
    denom = jnp.maximum(cnt, 1.0)
    mu = t[1:4] / denom
    sec = t[4:10] / denom
    mean_ref[...] = mu
    mx, my, mz = mu[0:1], mu[1:2], mu[2:3]
    c00 = sec[0:1] - mx * mx
    c01 = sec[1:2] - mx * my
    c02 = sec[2:3] - mx * mz
    c11 = sec[3:4] - my * my
    c12 = sec[4:5] - my * mz
    c22 = sec[5:6] - mz * mz
    cov_ref[...] = jnp.concatenate(
        [c00, c01, c02, c01, c11, c12, c02, c12, c22], axis=0)


def _finalize(parts3):
    return pl.pallas_call(
        _fin_body,
        out_shape=[jax.ShapeDtypeStruct((3, V), jnp.float32),
                   jax.ShapeDtypeStruct((9, V), jnp.float32)],
    )(parts3)


def kernel(x):
    outs = _depad(x)
    planes, params = outs[:3 * NSTR], outs[3 * NSTR]

    parts = _make_scatter(NSTR * planes[0].shape[0])(*planes, params)

    mean_t, cov_t = _finalize(parts.reshape(NW, NF, V))
    means = mean_t.T
    covs = cov_t.T.reshape(V, 3, 3)
    return means, covs
